# Initial kernel scaffold; baseline (speedup 1.0000x reference)
import jax
import jax.numpy as jnp
from jax.experimental import pallas as pl


def kernel(score_norm_data, W1, W3, W2, gate_W, A1, B1, A3, B3, A2, B2):
    raise NotImplementedError("write your pallas kernel here")


# speedup vs baseline: 3.4419x; 3.4419x over previous
import jax
import jax.numpy as jnp
from jax.experimental import pallas as pl

D = 768
FF = 2048
E = 8
R = 8
TB = 256


def _moe_kernel(x_ref, w1t_ref, w3t_ref, w2t_ref, gwt_ref,
                a1t_ref, b1t_ref, a3t_ref, b3t_ref, a2_ref, b2r_ref,
                out_ref):
    x = x_ref[...]

    def dot(a, b, ca=1, cb=0):
        return jax.lax.dot_general(
            a, b, (((ca,), (cb,)), ((), ())),
            preferred_element_type=jnp.float32)

    logits = dot(x, gwt_ref[...])
    iota = jax.lax.broadcasted_iota(jnp.int32, logits.shape, 1)
    m1 = jnp.max(logits, axis=1, keepdims=True)
    i1 = jnp.min(jnp.where(logits == m1, iota, E), axis=1, keepdims=True)
    masked = jnp.where(iota == i1, -jnp.inf, logits)
    m2 = jnp.max(masked, axis=1, keepdims=True)
    i2 = jnp.min(jnp.where(masked == m2, iota, E), axis=1, keepdims=True)
    p1 = 1.0 / (1.0 + jnp.exp(m2 - m1))
    w = jnp.where(iota == i1, p1, 0.0) + jnp.where(iota == i2, 1.0 - p1, 0.0)

    c1 = dot(x, w1t_ref[...])
    c3 = dot(x, w3t_ref[...])
    u1 = dot(x, a1t_ref[...])
    u3 = dot(x, a3t_ref[...])

    acc = jnp.zeros_like(c1)
    v2s = []
    for e in range(E):
        l1 = dot(u1[:, e * R:(e + 1) * R], b1t_ref[e])
        l3 = dot(u3[:, e * R:(e + 1) * R], b3t_ref[e])
        w1e = c1 + l1
        w3e = c3 + l3
        s = (w1e * jax.nn.sigmoid(w1e)) * w3e * w[:, e:e + 1]
        acc = acc + s
        v2s.append(dot(s, a2_ref[e], 1, 1))
    v2 = jnp.concatenate(v2s, axis=1)

    out_ref[...] = dot(acc, w2t_ref[...]) + dot(v2, b2r_ref[...])


def kernel(score_norm_data, W1, W3, W2, gate_W, A1, B1, A3, B3, A2, B2):
    T = score_norm_data.shape[0]
    W1t = W1.T
    W3t = W3.T
    W2t = W2.T
    gWt = gate_W.T
    A1t = A1.reshape(E * R, D).T
    A3t = A3.reshape(E * R, D).T
    B1t = jnp.swapaxes(B1, 1, 2)
    B3t = jnp.swapaxes(B3, 1, 2)
    B2r = jnp.transpose(B2, (0, 2, 1)).reshape(E * R, D)

    full = lambda shape: pl.BlockSpec(shape, lambda i: (0,) * len(shape))
    return pl.pallas_call(
        _moe_kernel,
        grid=(T // TB,),
        in_specs=[
            pl.BlockSpec((TB, D), lambda i: (i, 0)),
            full((D, FF)), full((D, FF)), full((FF, D)), full((D, E)),
            full((D, E * R)), full((E, R, FF)),
            full((D, E * R)), full((E, R, FF)),
            full((E, R, FF)), full((E * R, D)),
        ],
        out_specs=pl.BlockSpec((TB, D), lambda i: (i, 0)),
        out_shape=jax.ShapeDtypeStruct((T, D), jnp.float32),
    )(score_norm_data, W1t, W3t, W2t, gWt, A1t, B1t, A3t, B3t, A2, B2r)


# speedup vs baseline: 4.8169x; 1.3995x over previous
import jax
import jax.numpy as jnp
from jax.experimental import pallas as pl

D = 768
FF = 2048
E = 8
R = 8
TB = 256


def _moe_kernel(x_ref, w1t_ref, w3t_ref, w2t_ref, gwt_ref,
                a1t_ref, b1c_ref, a3t_ref, b3c_ref, a2ct_ref, b2c_ref,
                out_ref):
    x = x_ref[...]

    def dot(a, b):
        return jax.lax.dot_general(
            a, b, (((1,), (0,)), ((), ())),
            preferred_element_type=jnp.float32)

    logits = dot(x, gwt_ref[...])
    iota = jax.lax.broadcasted_iota(jnp.int32, logits.shape, 1)
    m1 = jnp.max(logits, axis=1, keepdims=True)
    i1 = jnp.min(jnp.where(logits == m1, iota, E), axis=1, keepdims=True)
    masked = jnp.where(iota == i1, -jnp.inf, logits)
    m2 = jnp.max(masked, axis=1, keepdims=True)
    i2 = jnp.min(jnp.where(masked == m2, iota, E), axis=1, keepdims=True)
    p1 = 1.0 / (1.0 + jnp.exp(m2 - m1))
    p2 = 1.0 - p1

    c1 = dot(x, w1t_ref[...])
    c3 = dot(x, w3t_ref[...])
    u1 = dot(x, a1t_ref[...])
    u3 = dot(x, a3t_ref[...])

    ecol = jax.lax.broadcasted_iota(jnp.int32, (x.shape[0], E * R), 1) // R
    m1c = (ecol == i1).astype(jnp.float32)
    m2c = (ecol == i2).astype(jnp.float32)

    out2 = None
    acc = None
    for mc, p in ((m1c, p1), (m2c, p2)):
        l1 = dot(u1 * mc, b1c_ref[...])
        l3 = dot(u3 * mc, b3c_ref[...])
        w1e = c1 + l1
        w3e = c3 + l3
        s = (w1e * jax.nn.sigmoid(w1e)) * w3e * p
        y = dot(s, a2ct_ref[...]) * mc
        acc = s if acc is None else acc + s
        out2 = y if out2 is None else out2 + y
    out_ref[...] = dot(acc, w2t_ref[...]) + dot(out2, b2c_ref[...])


def kernel(score_norm_data, W1, W3, W2, gate_W, A1, B1, A3, B3, A2, B2):
    T = score_norm_data.shape[0]
    W1t = W1.T
    W3t = W3.T
    W2t = W2.T
    gWt = gate_W.T
    A1t = A1.reshape(E * R, D).T
    A3t = A3.reshape(E * R, D).T
    B1c = jnp.swapaxes(B1, 1, 2).reshape(E * R, FF)
    B3c = jnp.swapaxes(B3, 1, 2).reshape(E * R, FF)
    A2ct = A2.reshape(E * R, FF).T
    B2c = jnp.transpose(B2, (0, 2, 1)).reshape(E * R, D)

    full = lambda shape: pl.BlockSpec(shape, lambda i: (0,) * len(shape))
    return pl.pallas_call(
        _moe_kernel,
        grid=(T // TB,),
        in_specs=[
            pl.BlockSpec((TB, D), lambda i: (i, 0)),
            full((D, FF)), full((D, FF)), full((FF, D)), full((D, E)),
            full((D, E * R)), full((E * R, FF)),
            full((D, E * R)), full((E * R, FF)),
            full((FF, E * R)), full((E * R, D)),
        ],
        out_specs=pl.BlockSpec((TB, D), lambda i: (i, 0)),
        out_shape=jax.ShapeDtypeStruct((T, D), jnp.float32),
    )(score_norm_data, W1t, W3t, W2t, gWt, A1t, B1c, A3t, B3c, A2ct, B2c)


# speedup vs baseline: 6.5183x; 1.3532x over previous
import jax
import jax.numpy as jnp
from jax.experimental import pallas as pl

D = 768
FF = 2048
E = 8
R = 8
TB = 256


def _moe_kernel(x_ref, w1t_ref, w3t_ref, w2t_ref, gwt_ref,
                a1t_ref, b1c_ref, a3t_ref, b3c_ref, a2ct_ref, b2c_ref,
                out_ref):
    x = x_ref[...]

    def dot(a, b):
        return jax.lax.dot_general(
            a, b, (((1,), (0,)), ((), ())),
            preferred_element_type=jnp.float32)

    def bdot(a, b):
        return dot(a.astype(jnp.bfloat16), b)

    logits = dot(x, gwt_ref[...])
    iota = jax.lax.broadcasted_iota(jnp.int32, logits.shape, 1)
    m1 = jnp.max(logits, axis=1, keepdims=True)
    i1 = jnp.min(jnp.where(logits == m1, iota, E), axis=1, keepdims=True)
    masked = jnp.where(iota == i1, -jnp.inf, logits)
    m2 = jnp.max(masked, axis=1, keepdims=True)
    i2 = jnp.min(jnp.where(masked == m2, iota, E), axis=1, keepdims=True)
    p1 = 1.0 / (1.0 + jnp.exp(m2 - m1))
    p2 = 1.0 - p1

    xb = x.astype(jnp.bfloat16)
    c1 = dot(xb, w1t_ref[...])
    c3 = dot(xb, w3t_ref[...])
    u1 = dot(xb, a1t_ref[...])
    u3 = dot(xb, a3t_ref[...])

    ecol = jax.lax.broadcasted_iota(jnp.int32, (x.shape[0], E * R), 1) // R
    m1c = (ecol == i1).astype(jnp.float32)
    m2c = (ecol == i2).astype(jnp.float32)

    out2 = None
    acc = None
    for mc, p in ((m1c, p1), (m2c, p2)):
        l1 = bdot(u1 * mc, b1c_ref[...])
        l3 = bdot(u3 * mc, b3c_ref[...])
        w1e = c1 + l1
        w3e = c3 + l3
        s = (w1e * jax.nn.sigmoid(w1e)) * w3e * p
        y = bdot(s, a2ct_ref[...]) * mc
        acc = s if acc is None else acc + s
        out2 = y if out2 is None else out2 + y
    out_ref[...] = bdot(acc, w2t_ref[...]) + bdot(out2, b2c_ref[...])


def kernel(score_norm_data, W1, W3, W2, gate_W, A1, B1, A3, B3, A2, B2):
    T = score_norm_data.shape[0]
    bf = jnp.bfloat16
    W1t = W1.T.astype(bf)
    W3t = W3.T.astype(bf)
    W2t = W2.T.astype(bf)
    gWt = gate_W.T
    A1t = A1.reshape(E * R, D).T.astype(bf)
    A3t = A3.reshape(E * R, D).T.astype(bf)
    B1c = jnp.swapaxes(B1, 1, 2).reshape(E * R, FF).astype(bf)
    B3c = jnp.swapaxes(B3, 1, 2).reshape(E * R, FF).astype(bf)
    A2ct = A2.reshape(E * R, FF).T.astype(bf)
    B2c = jnp.transpose(B2, (0, 2, 1)).reshape(E * R, D).astype(bf)

    full = lambda shape: pl.BlockSpec(shape, lambda i: (0,) * len(shape))
    return pl.pallas_call(
        _moe_kernel,
        grid=(T // TB,),
        in_specs=[
            pl.BlockSpec((TB, D), lambda i: (i, 0)),
            full((D, FF)), full((D, FF)), full((FF, D)), full((D, E)),
            full((D, E * R)), full((E * R, FF)),
            full((D, E * R)), full((E * R, FF)),
            full((FF, E * R)), full((E * R, D)),
        ],
        out_specs=pl.BlockSpec((TB, D), lambda i: (i, 0)),
        out_shape=jax.ShapeDtypeStruct((T, D), jnp.float32),
    )(score_norm_data, W1t, W3t, W2t, gWt, A1t, B1c, A3t, B3c, A2ct, B2c)


# speedup vs baseline: 7.5365x; 1.1562x over previous
import jax
import jax.numpy as jnp
from jax.experimental import pallas as pl
from jax.experimental.pallas import tpu as pltpu

D = 768
FF = 2048
E = 8
R = 8
TB = 256


def _moe_kernel(x_ref, w1_ref, w3_ref, w2_ref, gw_ref, a1r_ref, a3r_ref,
                sm_ref, out_ref, w1s, w3s, w2s):
    @pl.when(pl.program_id(0) == 0)
    def _prep():
        w1s[...] = w1_ref[...].astype(jnp.bfloat16)
        w3s[...] = w3_ref[...].astype(jnp.bfloat16)
        w2s[...] = w2_ref[...].astype(jnp.bfloat16)

    x = x_ref[...]

    def fdot(a, b):
        return jax.lax.dot_general(
            a, b, (((1,), (1,)), ((), ())),
            preferred_element_type=jnp.float32)

    def tdot(a, b):
        return fdot(a.astype(jnp.bfloat16), b)

    def bdot(a, b):
        return jax.lax.dot_general(
            a.astype(jnp.bfloat16), b, (((1,), (0,)), ((), ())),
            preferred_element_type=jnp.float32)

    logits = fdot(x, gw_ref[...])
    iota = jax.lax.broadcasted_iota(jnp.int32, logits.shape, 1)
    m1 = jnp.max(logits, axis=1, keepdims=True)
    i1 = jnp.min(jnp.where(logits == m1, iota, E), axis=1, keepdims=True)
    masked = jnp.where(iota == i1, -jnp.inf, logits)
    m2 = jnp.max(masked, axis=1, keepdims=True)
    i2 = jnp.min(jnp.where(masked == m2, iota, E), axis=1, keepdims=True)
    p1 = 1.0 / (1.0 + jnp.exp(m2 - m1))
    p2 = 1.0 - p1

    xb = x.astype(jnp.bfloat16)
    c1 = tdot(xb, w1s[...])
    c3 = tdot(xb, w3s[...])
    u1 = fdot(x, a1r_ref[...])
    u3 = fdot(x, a3r_ref[...])

    b1c = sm_ref[0:E * R, :]
    b3c = sm_ref[E * R:2 * E * R, :]
    a2c = sm_ref[2 * E * R:3 * E * R, :]
    b2c = sm_ref[3 * E * R:4 * E * R, 0:D]

    ecol = jax.lax.broadcasted_iota(jnp.int32, (x.shape[0], E * R), 1) // R
    m1c = (ecol == i1).astype(jnp.float32)
    m2c = (ecol == i2).astype(jnp.float32)

    out2 = None
    acc = None
    for mc, p in ((m1c, p1), (m2c, p2)):
        l1 = bdot(u1 * mc, b1c)
        l3 = bdot(u3 * mc, b3c)
        w1e = c1 + l1
        w3e = c3 + l3
        s = (w1e * jax.nn.sigmoid(w1e)) * w3e * p
        y = tdot(s, a2c) * mc
        acc = s if acc is None else acc + s
        out2 = y if out2 is None else out2 + y
    out_ref[...] = tdot(acc, w2s[...]) + bdot(out2, b2c)


def kernel(score_norm_data, W1, W3, W2, gate_W, A1, B1, A3, B3, A2, B2):
    T = score_norm_data.shape[0]
    B1c = jnp.swapaxes(B1, 1, 2).reshape(E * R, FF)
    B3c = jnp.swapaxes(B3, 1, 2).reshape(E * R, FF)
    A2c = A2.reshape(E * R, FF)
    B2c = jnp.pad(jnp.transpose(B2, (0, 2, 1)).reshape(E * R, D),
                  ((0, 0), (0, FF - D)))
    SM = jnp.concatenate([B1c, B3c, A2c, B2c], axis=0).astype(jnp.bfloat16)
    A1r = A1.reshape(E * R, D)
    A3r = A3.reshape(E * R, D)

    full = lambda shape: pl.BlockSpec(shape, lambda i: (0,) * len(shape))
    return pl.pallas_call(
        _moe_kernel,
        grid=(T // TB,),
        in_specs=[
            pl.BlockSpec((TB, D), lambda i: (i, 0)),
            full((FF, D)), full((FF, D)), full((D, FF)), full((E, D)),
            full((E * R, D)), full((E * R, D)),
            full((4 * E * R, FF)),
        ],
        out_specs=pl.BlockSpec((TB, D), lambda i: (i, 0)),
        out_shape=jax.ShapeDtypeStruct((T, D), jnp.float32),
        scratch_shapes=[
            pltpu.VMEM((FF, D), jnp.bfloat16),
            pltpu.VMEM((FF, D), jnp.bfloat16),
            pltpu.VMEM((D, FF), jnp.bfloat16),
        ],
    )(score_norm_data, W1, W3, W2, gate_W, A1r, A3r, SM)


# speedup vs baseline: 7.8747x; 1.0449x over previous
import jax
import jax.numpy as jnp
from jax.experimental import pallas as pl
from jax.experimental.pallas import tpu as pltpu

D = 768
FF = 2048
E = 8
R = 8
TB = 512


def _moe_kernel(x_ref, w1_ref, w3_ref, w2_ref, gw_ref, a1r_ref, a3r_ref,
                sm_ref, out_ref, w1s, w3s, w2s):
    @pl.when(pl.program_id(0) == 0)
    def _prep():
        w1s[...] = w1_ref[...].astype(jnp.bfloat16)
        w3s[...] = w3_ref[...].astype(jnp.bfloat16)
        w2s[...] = w2_ref[...].astype(jnp.bfloat16)

    x = x_ref[...]

    def fdot(a, b):
        return jax.lax.dot_general(
            a, b, (((1,), (1,)), ((), ())),
            preferred_element_type=jnp.float32)

    def tdot(a, b):
        return fdot(a.astype(jnp.bfloat16), b)

    def bdot(a, b):
        return jax.lax.dot_general(
            a.astype(jnp.bfloat16), b, (((1,), (0,)), ((), ())),
            preferred_element_type=jnp.float32)

    logits = fdot(x, gw_ref[...])
    iota = jax.lax.broadcasted_iota(jnp.int32, logits.shape, 1)
    m1 = jnp.max(logits, axis=1, keepdims=True)
    i1 = jnp.min(jnp.where(logits == m1, iota, E), axis=1, keepdims=True)
    masked = jnp.where(iota == i1, -jnp.inf, logits)
    m2 = jnp.max(masked, axis=1, keepdims=True)
    i2 = jnp.min(jnp.where(masked == m2, iota, E), axis=1, keepdims=True)
    p1 = 1.0 / (1.0 + jnp.exp(m2 - m1))
    p2 = 1.0 - p1

    xb = x.astype(jnp.bfloat16)
    c1 = tdot(xb, w1s[...])
    c3 = tdot(xb, w3s[...])
    u1 = fdot(x, a1r_ref[...])
    u3 = fdot(x, a3r_ref[...])

    b1c = sm_ref[0:E * R, :]
    b3c = sm_ref[E * R:2 * E * R, :]
    a2c = sm_ref[2 * E * R:3 * E * R, :]
    b2c = sm_ref[3 * E * R:4 * E * R, 0:D]

    ecol = jax.lax.broadcasted_iota(jnp.int32, (x.shape[0], E * R), 1) // R
    m1c = (ecol == i1).astype(jnp.float32)
    m2c = (ecol == i2).astype(jnp.float32)

    out2 = None
    acc = None
    for mc, p in ((m1c, p1), (m2c, p2)):
        l1 = bdot(u1 * mc, b1c)
        l3 = bdot(u3 * mc, b3c)
        w1e = c1 + l1
        w3e = c3 + l3
        s = (w1e * jax.nn.sigmoid(w1e)) * w3e * p
        y = tdot(s, a2c) * mc
        acc = s if acc is None else acc + s
        out2 = y if out2 is None else out2 + y
    out_ref[...] = tdot(acc, w2s[...]) + bdot(out2, b2c)


def kernel(score_norm_data, W1, W3, W2, gate_W, A1, B1, A3, B3, A2, B2):
    T = score_norm_data.shape[0]
    B1c = jnp.swapaxes(B1, 1, 2).reshape(E * R, FF)
    B3c = jnp.swapaxes(B3, 1, 2).reshape(E * R, FF)
    A2c = A2.reshape(E * R, FF)
    B2c = jnp.pad(jnp.transpose(B2, (0, 2, 1)).reshape(E * R, D),
                  ((0, 0), (0, FF - D)))
    SM = jnp.concatenate([B1c, B3c, A2c, B2c], axis=0).astype(jnp.bfloat16)
    A1r = A1.reshape(E * R, D)
    A3r = A3.reshape(E * R, D)

    full = lambda shape: pl.BlockSpec(shape, lambda i: (0,) * len(shape))
    return pl.pallas_call(
        _moe_kernel,
        grid=(T // TB,),
        in_specs=[
            pl.BlockSpec((TB, D), lambda i: (i, 0)),
            full((FF, D)), full((FF, D)), full((D, FF)), full((E, D)),
            full((E * R, D)), full((E * R, D)),
            full((4 * E * R, FF)),
        ],
        out_specs=pl.BlockSpec((TB, D), lambda i: (i, 0)),
        out_shape=jax.ShapeDtypeStruct((T, D), jnp.float32),
        scratch_shapes=[
            pltpu.VMEM((FF, D), jnp.bfloat16),
            pltpu.VMEM((FF, D), jnp.bfloat16),
            pltpu.VMEM((D, FF), jnp.bfloat16),
        ],
    )(score_norm_data, W1, W3, W2, gate_W, A1r, A3r, SM)


# speedup vs baseline: 7.9713x; 1.0123x over previous
import jax
import jax.numpy as jnp
from jax.experimental import pallas as pl
from jax.experimental.pallas import tpu as pltpu

D = 768
FF = 2048
E = 8
R = 8
TB = 512


def _moe_kernel(x_ref, w1_ref, w3_ref, w2_ref, gw_ref, a1r_ref, a3r_ref,
                sm_ref, out_ref, w1s, w3s, w2s):
    w1s[...] = w1_ref[...].astype(jnp.bfloat16)
    w3s[...] = w3_ref[...].astype(jnp.bfloat16)
    w2s[...] = w2_ref[...].astype(jnp.bfloat16)

    x = x_ref[...]

    def fdot(a, b):
        return jax.lax.dot_general(
            a, b, (((1,), (1,)), ((), ())),
            preferred_element_type=jnp.float32)

    def tdot(a, b):
        return fdot(a.astype(jnp.bfloat16), b)

    def bdot(a, b):
        return jax.lax.dot_general(
            a.astype(jnp.bfloat16), b, (((1,), (0,)), ((), ())),
            preferred_element_type=jnp.float32)

    logits = fdot(x, gw_ref[...])
    iota = jax.lax.broadcasted_iota(jnp.int32, logits.shape, 1)
    m1 = jnp.max(logits, axis=1, keepdims=True)
    i1 = jnp.min(jnp.where(logits == m1, iota, E), axis=1, keepdims=True)
    masked = jnp.where(iota == i1, -jnp.inf, logits)
    m2 = jnp.max(masked, axis=1, keepdims=True)
    i2 = jnp.min(jnp.where(masked == m2, iota, E), axis=1, keepdims=True)
    p1 = 1.0 / (1.0 + jnp.exp(m2 - m1))
    p2 = 1.0 - p1

    xb = x.astype(jnp.bfloat16)
    c1 = tdot(xb, w1s[...])
    c3 = tdot(xb, w3s[...])
    u1 = fdot(x, a1r_ref[...])
    u3 = fdot(x, a3r_ref[...])

    b1c = sm_ref[0:E * R, :]
    b3c = sm_ref[E * R:2 * E * R, :]
    a2c = sm_ref[2 * E * R:3 * E * R, :]
    b2c = sm_ref[3 * E * R:4 * E * R, 0:D]

    ecol = jax.lax.broadcasted_iota(jnp.int32, (x.shape[0], E * R), 1) // R
    m1c = (ecol == i1).astype(jnp.float32)
    m2c = (ecol == i2).astype(jnp.float32)

    out2 = None
    acc = None
    for mc, p in ((m1c, p1), (m2c, p2)):
        l1 = bdot(u1 * mc, b1c)
        l3 = bdot(u3 * mc, b3c)
        w1e = c1 + l1
        w3e = c3 + l3
        s = (w1e * jax.nn.sigmoid(w1e)) * w3e * p
        y = tdot(s, a2c) * mc
        acc = s if acc is None else acc + s
        out2 = y if out2 is None else out2 + y
    out_ref[...] = tdot(acc, w2s[...]) + bdot(out2, b2c)


def kernel(score_norm_data, W1, W3, W2, gate_W, A1, B1, A3, B3, A2, B2):
    T = score_norm_data.shape[0]
    B1c = jnp.swapaxes(B1, 1, 2).reshape(E * R, FF)
    B3c = jnp.swapaxes(B3, 1, 2).reshape(E * R, FF)
    A2c = A2.reshape(E * R, FF)
    B2c = jnp.pad(jnp.transpose(B2, (0, 2, 1)).reshape(E * R, D),
                  ((0, 0), (0, FF - D)))
    SM = jnp.concatenate([B1c, B3c, A2c, B2c], axis=0).astype(jnp.bfloat16)
    A1r = A1.reshape(E * R, D)
    A3r = A3.reshape(E * R, D)

    full = lambda shape: pl.BlockSpec(shape, lambda i: (0,) * len(shape))
    return pl.pallas_call(
        _moe_kernel,
        grid=(T // TB,),
        in_specs=[
            pl.BlockSpec((TB, D), lambda i: (i, 0)),
            full((FF, D)), full((FF, D)), full((D, FF)), full((E, D)),
            full((E * R, D)), full((E * R, D)),
            full((4 * E * R, FF)),
        ],
        out_specs=pl.BlockSpec((TB, D), lambda i: (i, 0)),
        out_shape=jax.ShapeDtypeStruct((T, D), jnp.float32),
        scratch_shapes=[
            pltpu.VMEM((FF, D), jnp.bfloat16),
            pltpu.VMEM((FF, D), jnp.bfloat16),
            pltpu.VMEM((D, FF), jnp.bfloat16),
        ],
        compiler_params=pltpu.CompilerParams(
            dimension_semantics=("parallel",)),
    )(score_norm_data, W1, W3, W2, gate_W, A1r, A3r, SM)


# speedup vs baseline: 8.8217x; 1.1067x over previous
import jax
import jax.numpy as jnp
from jax.experimental import pallas as pl
from jax.experimental.pallas import tpu as pltpu

D = 768
FF = 2048
E = 8
R = 8
ER = E * R
FB = 512


def _moe_kernel(x_ref, w1_ref, w3_ref, w2_ref, gw_ref, a1r_ref, a3r_ref,
                sm_ref, b2c_ref, out_ref, xbs, um, pp, msk, yac):
    j = pl.program_id(0)
    bf = jnp.bfloat16

    def fdot(a, b):
        return jax.lax.dot_general(
            a, b, (((1,), (1,)), ((), ())),
            preferred_element_type=jnp.float32)

    def bdot(a, b):
        return jax.lax.dot_general(
            a, b, (((1,), (0,)), ((), ())),
            preferred_element_type=jnp.float32)

    @pl.when(j == 0)
    def _prep():
        x = x_ref[...]
        logits = fdot(x, gw_ref[...])
        iota = jax.lax.broadcasted_iota(jnp.int32, logits.shape, 1)
        m1 = jnp.max(logits, axis=1, keepdims=True)
        i1 = jnp.min(jnp.where(logits == m1, iota, E), axis=1, keepdims=True)
        masked = jnp.where(iota == i1, -jnp.inf, logits)
        m2 = jnp.max(masked, axis=1, keepdims=True)
        i2 = jnp.min(jnp.where(masked == m2, iota, E), axis=1, keepdims=True)
        p1 = 1.0 / (1.0 + jnp.exp(m2 - m1))
        ecol = jax.lax.broadcasted_iota(jnp.int32, (x.shape[0], ER), 1) // R
        m1c = (ecol == i1).astype(jnp.float32)
        m2c = (ecol == i2).astype(jnp.float32)
        u1 = fdot(x, a1r_ref[...])
        u3 = fdot(x, a3r_ref[...])
        xbs[...] = x.astype(bf)
        um[:, 0 * ER:1 * ER] = (u1 * m1c).astype(bf)
        um[:, 1 * ER:2 * ER] = (u3 * m1c).astype(bf)
        um[:, 2 * ER:3 * ER] = (u1 * m2c).astype(bf)
        um[:, 3 * ER:4 * ER] = (u3 * m2c).astype(bf)
        pp[...] = jnp.broadcast_to(p1, pp.shape)
        msk[:, 0:ER] = m1c
        msk[:, ER:2 * ER] = m2c
        yac[...] = jnp.zeros_like(yac)
        out_ref[...] = jnp.zeros_like(out_ref)

    xb = xbs[...]
    w1b = w1_ref[...].astype(bf)
    w3b = w3_ref[...].astype(bf)
    w2b = w2_ref[...].astype(bf)
    c1 = fdot(xb, w1b)
    c3 = fdot(xb, w3b)
    b1c = sm_ref[0:ER, :]
    b3c = sm_ref[ER:2 * ER, :]
    a2c = sm_ref[2 * ER:3 * ER, :]

    p1 = pp[:, 0:1]
    acc = None
    for k in range(2):
        l1 = bdot(um[:, (2 * k) * ER:(2 * k + 1) * ER], b1c)
        l3 = bdot(um[:, (2 * k + 1) * ER:(2 * k + 2) * ER], b3c)
        w1e = c1 + l1
        w3e = c3 + l3
        p = p1 if k == 0 else 1.0 - p1
        s = (w1e * jax.nn.sigmoid(w1e)) * w3e * p
        yac[:, k * ER:(k + 1) * ER] += fdot(s.astype(bf), a2c)
        acc = s if acc is None else acc + s
    out_ref[...] += fdot(acc.astype(bf), w2b)

    @pl.when(j == pl.num_programs(0) - 1)
    def _fin():
        y = (yac[:, 0:ER] * msk[:, 0:ER]
             + yac[:, ER:2 * ER] * msk[:, ER:2 * ER])
        out_ref[...] += bdot(y.astype(bf), b2c_ref[...])


def kernel(score_norm_data, W1, W3, W2, gate_W, A1, B1, A3, B3, A2, B2):
    T = score_norm_data.shape[0]
    B1c = jnp.swapaxes(B1, 1, 2).reshape(ER, FF)
    B3c = jnp.swapaxes(B3, 1, 2).reshape(ER, FF)
    A2c = A2.reshape(ER, FF)
    SM = jnp.concatenate([B1c, B3c, A2c], axis=0).astype(jnp.bfloat16)
    B2c = jnp.transpose(B2, (0, 2, 1)).reshape(ER, D).astype(jnp.bfloat16)
    A1r = A1.reshape(ER, D)
    A3r = A3.reshape(ER, D)

    full = lambda shape: pl.BlockSpec(shape, lambda j: (0,) * len(shape))
    return pl.pallas_call(
        _moe_kernel,
        grid=(FF // FB,),
        in_specs=[
            full((T, D)),
            pl.BlockSpec((FB, D), lambda j: (j, 0)),
            pl.BlockSpec((FB, D), lambda j: (j, 0)),
            pl.BlockSpec((D, FB), lambda j: (0, j)),
            full((E, D)),
            full((ER, D)), full((ER, D)),
            pl.BlockSpec((3 * ER, FB), lambda j: (0, j)),
            full((ER, D)),
        ],
        out_specs=full((T, D)),
        out_shape=jax.ShapeDtypeStruct((T, D), jnp.float32),
        scratch_shapes=[
            pltpu.VMEM((T, D), jnp.bfloat16),
            pltpu.VMEM((T, 4 * ER), jnp.bfloat16),
            pltpu.VMEM((T, 128), jnp.float32),
            pltpu.VMEM((T, 2 * ER), jnp.float32),
            pltpu.VMEM((T, 2 * ER), jnp.float32),
        ],
    )(score_norm_data, W1, W3, W2, gate_W, A1r, A3r, SM, B2c)


# speedup vs baseline: 9.3594x; 1.0610x over previous
import jax
import jax.numpy as jnp
from jax.experimental import pallas as pl
from jax.experimental.pallas import tpu as pltpu

D = 768
FF = 2048
E = 8
R = 8
ER = E * R
FB = 256


def _moe_kernel(x_ref, w1_ref, w3_ref, w2_ref, gw_ref, a1r_ref, a3r_ref,
                sm_ref, b2c_ref, out_ref, xbs, um, pp, msk, yac):
    j = pl.program_id(0)
    bf = jnp.bfloat16

    def fdot(a, b):
        return jax.lax.dot_general(
            a, b, (((1,), (1,)), ((), ())),
            preferred_element_type=jnp.float32)

    def bdot(a, b):
        return jax.lax.dot_general(
            a, b, (((1,), (0,)), ((), ())),
            preferred_element_type=jnp.float32)

    @pl.when(j == 0)
    def _prep():
        x = x_ref[...]
        logits = fdot(x, gw_ref[...])
        iota = jax.lax.broadcasted_iota(jnp.int32, logits.shape, 1)
        m1 = jnp.max(logits, axis=1, keepdims=True)
        i1 = jnp.min(jnp.where(logits == m1, iota, E), axis=1, keepdims=True)
        masked = jnp.where(iota == i1, -jnp.inf, logits)
        m2 = jnp.max(masked, axis=1, keepdims=True)
        i2 = jnp.min(jnp.where(masked == m2, iota, E), axis=1, keepdims=True)
        p1 = 1.0 / (1.0 + jnp.exp(m2 - m1))
        ecol = jax.lax.broadcasted_iota(jnp.int32, (x.shape[0], ER), 1) // R
        m1c = (ecol == i1).astype(jnp.float32)
        m2c = (ecol == i2).astype(jnp.float32)
        u1 = fdot(x, a1r_ref[...])
        u3 = fdot(x, a3r_ref[...])
        xbs[...] = x.astype(bf)
        um[:, 0 * ER:1 * ER] = (u1 * m1c).astype(bf)
        um[:, 1 * ER:2 * ER] = (u3 * m1c).astype(bf)
        um[:, 2 * ER:3 * ER] = (u1 * m2c).astype(bf)
        um[:, 3 * ER:4 * ER] = (u3 * m2c).astype(bf)
        pp[...] = jnp.broadcast_to(p1, pp.shape)
        msk[:, 0:ER] = m1c
        msk[:, ER:2 * ER] = m2c
        yac[...] = jnp.zeros_like(yac)
        out_ref[...] = jnp.zeros_like(out_ref)

    xb = xbs[...]
    w1b = w1_ref[...].astype(bf)
    w3b = w3_ref[...].astype(bf)
    w2b = w2_ref[...].astype(bf)
    c1 = fdot(xb, w1b)
    c3 = fdot(xb, w3b)
    b1c = sm_ref[0:ER, :]
    b3c = sm_ref[ER:2 * ER, :]
    a2c = sm_ref[2 * ER:3 * ER, :]

    p1 = pp[:, 0:1]
    acc = None
    for k in range(2):
        l1 = bdot(um[:, (2 * k) * ER:(2 * k + 1) * ER], b1c)
        l3 = bdot(um[:, (2 * k + 1) * ER:(2 * k + 2) * ER], b3c)
        w1e = c1 + l1
        w3e = c3 + l3
        p = p1 if k == 0 else 1.0 - p1
        s = (w1e * jax.nn.sigmoid(w1e)) * w3e * p
        yac[:, k * ER:(k + 1) * ER] += fdot(s.astype(bf), a2c)
        acc = s if acc is None else acc + s
    out_ref[...] += fdot(acc.astype(bf), w2b)

    @pl.when(j == pl.num_programs(0) - 1)
    def _fin():
        y = (yac[:, 0:ER] * msk[:, 0:ER]
             + yac[:, ER:2 * ER] * msk[:, ER:2 * ER])
        out_ref[...] += bdot(y.astype(bf), b2c_ref[...])


def kernel(score_norm_data, W1, W3, W2, gate_W, A1, B1, A3, B3, A2, B2):
    T = score_norm_data.shape[0]
    B1c = jnp.swapaxes(B1, 1, 2).reshape(ER, FF)
    B3c = jnp.swapaxes(B3, 1, 2).reshape(ER, FF)
    A2c = A2.reshape(ER, FF)
    SM = jnp.concatenate([B1c, B3c, A2c], axis=0).astype(jnp.bfloat16)
    B2c = jnp.transpose(B2, (0, 2, 1)).reshape(ER, D).astype(jnp.bfloat16)
    A1r = A1.reshape(ER, D)
    A3r = A3.reshape(ER, D)

    full = lambda shape: pl.BlockSpec(shape, lambda j: (0,) * len(shape))
    return pl.pallas_call(
        _moe_kernel,
        grid=(FF // FB,),
        in_specs=[
            full((T, D)),
            pl.BlockSpec((FB, D), lambda j: (j, 0)),
            pl.BlockSpec((FB, D), lambda j: (j, 0)),
            pl.BlockSpec((D, FB), lambda j: (0, j)),
            full((E, D)),
            full((ER, D)), full((ER, D)),
            pl.BlockSpec((3 * ER, FB), lambda j: (0, j)),
            full((ER, D)),
        ],
        out_specs=full((T, D)),
        out_shape=jax.ShapeDtypeStruct((T, D), jnp.float32),
        scratch_shapes=[
            pltpu.VMEM((T, D), jnp.bfloat16),
            pltpu.VMEM((T, 4 * ER), jnp.bfloat16),
            pltpu.VMEM((T, 128), jnp.float32),
            pltpu.VMEM((T, 2 * ER), jnp.float32),
            pltpu.VMEM((T, 2 * ER), jnp.float32),
        ],
    )(score_norm_data, W1, W3, W2, gate_W, A1r, A3r, SM, B2c)
